# baseline (speedup 1.0000x reference)
import dataclasses
import functools

import jax
import jax.numpy as jnp
from jax import lax
from jax.experimental import pallas as pl
from jax.experimental.pallas import tpu as pltpu
from jax.experimental.pallas import tpu_sc as plsc

N, C, H, W = 4, 96, 384, 384
CP = 128
NPIX = N * H * W
HB = 64
NC, NS, L = 2, 16, 16
NWORK = NC * NS
PPW = NPIX // NWORK
G = 128
CHUNKS = PPW // G


def _prep_body(flow_ref, idx_ref, w_ref, mask_ref):
    n = pl.program_id(0)
    hb = pl.program_id(1)
    fx = flow_ref[0, 0]
    fy = flow_ref[0, 1]
    wc = lax.broadcasted_iota(jnp.int32, (HB, W), 1).astype(jnp.float32)
    hc = (lax.broadcasted_iota(jnp.int32, (HB, W), 0) + hb * HB).astype(jnp.float32)
    gridx = wc + fx
    gridy = hc + fy
    gx = 2.0 * gridx / (W - 1) - 1.0
    gy = 2.0 * gridy / (H - 1) - 1.0
    ix = ((gx + 1.0) * W - 1.0) / 2.0
    iy = ((gy + 1.0) * H - 1.0) / 2.0
    ix0 = jnp.floor(ix)
    iy0 = jnp.floor(iy)
    ix1 = ix0 + 1.0
    iy1 = iy0 + 1.0
    wx1 = ix - ix0
    wx0 = 1.0 - wx1
    wy1 = iy - iy0
    wy0 = 1.0 - wy1
    mask = (jnp.abs(gx) <= 1.0) & (jnp.abs(gy) <= 1.0)
    mask_ref[...] = mask.astype(jnp.int32)
    corners = [(iy0, ix0, wy0 * wx0), (iy0, ix1, wy0 * wx1),
               (iy1, ix0, wy1 * wx0), (iy1, ix1, wy1 * wx1)]
    for k, (iyc, ixc, wgt) in enumerate(corners):
        valid = (ixc >= 0) & (ixc <= W - 1) & (iyc >= 0) & (iyc <= H - 1)
        ixs = jnp.clip(ixc, 0, W - 1).astype(jnp.int32)
        iys = jnp.clip(iyc, 0, H - 1).astype(jnp.int32)
        p = (n * H + iys) * W + ixs
        p = jnp.clip(p, 0, NPIX - 1)
        idx_ref[k] = p
        w_ref[k] = wgt * valid.astype(jnp.float32)


_prep = pl.pallas_call(
    _prep_body,
    grid=(N, H // HB),
    in_specs=[pl.BlockSpec((1, 2, HB, W), lambda n, hb: (n, 0, hb, 0))],
    out_specs=[
        pl.BlockSpec((4, HB, W), lambda n, hb: (0, n * (H // HB) + hb, 0)),
        pl.BlockSpec((4, HB, W), lambda n, hb: (0, n * (H // HB) + hb, 0)),
        pl.BlockSpec((HB, W), lambda n, hb: (n * (H // HB) + hb, 0)),
    ],
    out_shape=[
        jax.ShapeDtypeStruct((4, N * H, W), jnp.int32),
        jax.ShapeDtypeStruct((4, N * H, W), jnp.float32),
        jax.ShapeDtypeStruct((N * H, W), jnp.int32),
    ],
)


@functools.cache
def _make_sc_warp():
    cp = pltpu.CompilerParams()
    if "needs_layout_passes" in pltpu.CompilerParams.__dataclass_fields__:
        cp = dataclasses.replace(cp, needs_layout_passes=False)
    if "use_tc_tiling_on_sc" in pltpu.CompilerParams.__dataclass_fields__:
        cp = dataclasses.replace(cp, use_tc_tiling_on_sc=False)

    @functools.partial(
        pl.kernel,
        compiler_params=cp,
        out_type=jax.ShapeDtypeStruct((NPIX, CP), jnp.float32),
        mesh=plsc.VectorSubcoreMesh(core_axis_name="c", subcore_axis_name="s",
                                    num_cores=NC, num_subcores=NS),
        scratch_types=[
            pltpu.VMEM((2, 4, G), jnp.int32),
            pltpu.VMEM((2, 4, G), jnp.float32),
            pltpu.VMEM((2, 4, G, C), jnp.bfloat16),
            pltpu.VMEM((2, G, CP), jnp.float32),
            pltpu.SemaphoreType.DMA,
            pltpu.SemaphoreType.DMA,
            pltpu.SemaphoreType.DMA,
            pltpu.SemaphoreType.DMA,
            pltpu.SemaphoreType.DMA,
            pltpu.SemaphoreType.DMA,
        ],
    )
    def _sc_warp(xt_hbm, idx_hbm, w_hbm, out_hbm, ibuf, wbuf, rows, obuf,
                 g0, g1, i0, i1, o0, o1):
        cid = lax.axis_index("c")
        sid = lax.axis_index("s")
        wid = cid * NS + sid
        gsem = (g0, g1)
        isem = (i0, i1)
        osem = (o0, o1)

        def issue_iw(j, p):
            base = wid * PPW + j * G
            pltpu.async_copy(idx_hbm.at[:, pl.ds(base, G)], ibuf.at[p], isem[p])
            pltpu.async_copy(w_hbm.at[:, pl.ds(base, G)], wbuf.at[p], isem[p])

        def drain_iw(p):
            pltpu.make_async_copy(idx_hbm.at[:, pl.ds(0, G)], ibuf.at[p],
                                  isem[p]).wait()
            pltpu.make_async_copy(w_hbm.at[:, pl.ds(0, G)], wbuf.at[p],
                                  isem[p]).wait()

        def issue_gathers(p):
            for k in range(4):
                pltpu.async_copy(xt_hbm.at[ibuf.at[p, k]], rows.at[p, k],
                                 gsem[p])

        def drain_gathers(p):
            for k in range(4):
                pltpu.make_async_copy(xt_hbm.at[pl.ds(0, G)], rows.at[p, k],
                                      gsem[p]).wait()

        def issue_out(j, p):
            base = wid * PPW + j * G
            pltpu.async_copy(obuf.at[p], out_hbm.at[pl.ds(base, G)], osem[p])

        def drain_out(p):
            pltpu.make_async_copy(obuf.at[p], out_hbm.at[pl.ds(0, G)],
                                  osem[p]).wait()

        def blend(p):
            iota = lax.iota(jnp.int32, L)

            @pl.loop(0, G)
            def _pix(i):
                isplat = lax.broadcast(i, (L,))
                wb = [plsc.load_gather(wbuf.at[p, k], [isplat])
                      for k in range(4)]
                for cb in range(0, C, 2 * L):
                    acc_e = None
                    acc_o = None
                    for k in range(4):
                        rv = rows[p, k, i, pl.ds(cb, 2 * L)]
                        e, o = plsc.unpack(rv, format=plsc.PackFormat.INTERLEAVED,
                                           preferred_element_type=jnp.float32)
                        if acc_e is None:
                            acc_e = wb[k] * e
                            acc_o = wb[k] * o
                        else:
                            acc_e = acc_e + wb[k] * e
                            acc_o = acc_o + wb[k] * o
                    plsc.store_scatter(obuf.at[p], [isplat, cb + 2 * iota], acc_e)
                    plsc.store_scatter(obuf.at[p], [isplat, cb + 1 + 2 * iota], acc_o)

        issue_iw(0, 0)
        issue_iw(1, 1)
        drain_iw(0)
        issue_gathers(0)

        @pl.loop(0, CHUNKS, step=2)
        def _chunk(j2):
            for p in (0, 1):
                j = j2 + p
                q = 1 - p
                drain_gathers(p)

                @pl.when(j + 1 < CHUNKS)
                def _():
                    drain_iw(q)
                    issue_gathers(q)

                @pl.when(j >= 2)
                def _():
                    drain_out(p)

                blend(p)
                issue_out(j, p)

                @pl.when(j + 2 < CHUNKS)
                def _():
                    issue_iw(j + 2, p)

        drain_out(0)
        drain_out(1)

    return _sc_warp


def kernel(x, flow):
    assert x.shape == (N, C, H, W) and flow.shape == (N, 2, H, W)
    idx, wts, maskarr = _prep(flow)
    xtb = jnp.transpose(x, (0, 2, 3, 1)).astype(jnp.bfloat16)
    out_flat = _make_sc_warp()(xtb.reshape(NPIX, C), idx.reshape(4, NPIX),
                               wts.reshape(4, NPIX))
    warped = jnp.transpose(out_flat.reshape(N, H, W, CP)[..., :C], (0, 3, 1, 2))
    mask = maskarr.reshape(N, 1, H, W).astype(jnp.bool_)
    return warped, mask


# speedup vs baseline: 1.3060x; 1.3060x over previous
import dataclasses
import functools

import jax
import jax.numpy as jnp
from jax import lax
from jax.experimental import pallas as pl
from jax.experimental.pallas import tpu as pltpu
from jax.experimental.pallas import tpu_sc as plsc

N, C, H, W = 4, 96, 384, 384
CP = 128
NPIX = N * H * W
HB = 64
NC, NS, L = 2, 16, 16
NWORK = NC * NS
PPW = NPIX // NWORK
G = 96
CHUNKS = PPW // G


def _prep_body(flow_ref, idx_ref, w_ref, mask_ref):
    n = pl.program_id(0)
    hb = pl.program_id(1)
    fx = flow_ref[0, 0]
    fy = flow_ref[0, 1]
    wc = lax.broadcasted_iota(jnp.int32, (HB, W), 1).astype(jnp.float32)
    hc = (lax.broadcasted_iota(jnp.int32, (HB, W), 0) + hb * HB).astype(jnp.float32)
    gridx = wc + fx
    gridy = hc + fy
    gx = 2.0 * gridx / (W - 1) - 1.0
    gy = 2.0 * gridy / (H - 1) - 1.0
    ix = ((gx + 1.0) * W - 1.0) / 2.0
    iy = ((gy + 1.0) * H - 1.0) / 2.0
    ix0 = jnp.floor(ix)
    iy0 = jnp.floor(iy)
    ix1 = ix0 + 1.0
    iy1 = iy0 + 1.0
    wx1 = ix - ix0
    wx0 = 1.0 - wx1
    wy1 = iy - iy0
    wy0 = 1.0 - wy1
    mask = (jnp.abs(gx) <= 1.0) & (jnp.abs(gy) <= 1.0)
    mask_ref[...] = mask.astype(jnp.int32)
    corners = [(iy0, ix0, wy0 * wx0), (iy0, ix1, wy0 * wx1),
               (iy1, ix0, wy1 * wx0), (iy1, ix1, wy1 * wx1)]
    for k, (iyc, ixc, wgt) in enumerate(corners):
        valid = (ixc >= 0) & (ixc <= W - 1) & (iyc >= 0) & (iyc <= H - 1)
        ixs = jnp.clip(ixc, 0, W - 1).astype(jnp.int32)
        iys = jnp.clip(iyc, 0, H - 1).astype(jnp.int32)
        p = (n * H + iys) * W + ixs
        p = jnp.clip(p, 0, NPIX - 1)
        idx_ref[k] = p
        w_ref[k] = wgt * valid.astype(jnp.float32)


_prep = pl.pallas_call(
    _prep_body,
    grid=(N, H // HB),
    in_specs=[pl.BlockSpec((1, 2, HB, W), lambda n, hb: (n, 0, hb, 0))],
    out_specs=[
        pl.BlockSpec((4, HB, W), lambda n, hb: (0, n * (H // HB) + hb, 0)),
        pl.BlockSpec((4, HB, W), lambda n, hb: (0, n * (H // HB) + hb, 0)),
        pl.BlockSpec((HB, W), lambda n, hb: (n * (H // HB) + hb, 0)),
    ],
    out_shape=[
        jax.ShapeDtypeStruct((4, N * H, W), jnp.int32),
        jax.ShapeDtypeStruct((4, N * H, W), jnp.float32),
        jax.ShapeDtypeStruct((N * H, W), jnp.int32),
    ],
)


@functools.cache
def _make_sc_warp():
    cp = pltpu.CompilerParams()
    if "needs_layout_passes" in pltpu.CompilerParams.__dataclass_fields__:
        cp = dataclasses.replace(cp, needs_layout_passes=False)
    if "use_tc_tiling_on_sc" in pltpu.CompilerParams.__dataclass_fields__:
        cp = dataclasses.replace(cp, use_tc_tiling_on_sc=False)

    @functools.partial(
        pl.kernel,
        compiler_params=cp,
        out_type=jax.ShapeDtypeStruct((NPIX, CP), jnp.float32),
        mesh=plsc.VectorSubcoreMesh(core_axis_name="c", subcore_axis_name="s",
                                    num_cores=NC, num_subcores=NS),
        scratch_types=[
            pltpu.VMEM((2, 4, G), jnp.int32),
            pltpu.VMEM((2, 4, G), jnp.float32),
            pltpu.VMEM((2, 4, G, CP), jnp.float32),
            pltpu.VMEM((2, G, CP), jnp.float32),
            pltpu.SemaphoreType.DMA,
            pltpu.SemaphoreType.DMA,
            pltpu.SemaphoreType.DMA,
            pltpu.SemaphoreType.DMA,
            pltpu.SemaphoreType.DMA,
            pltpu.SemaphoreType.DMA,
        ],
    )
    def _sc_warp(xt_hbm, idx_hbm, w_hbm, out_hbm, ibuf, wbuf, rows, obuf,
                 g0, g1, i0, i1, o0, o1):
        cid = lax.axis_index("c")
        sid = lax.axis_index("s")
        wid = cid * NS + sid
        gsem = (g0, g1)
        isem = (i0, i1)
        osem = (o0, o1)

        def issue_iw(j, p):
            base = wid * PPW + j * G
            pltpu.async_copy(idx_hbm.at[:, pl.ds(base, G)], ibuf.at[p], isem[p])
            pltpu.async_copy(w_hbm.at[:, pl.ds(base, G)], wbuf.at[p], isem[p])

        def drain_iw(p):
            pltpu.make_async_copy(idx_hbm.at[:, pl.ds(0, G)], ibuf.at[p],
                                  isem[p]).wait()
            pltpu.make_async_copy(w_hbm.at[:, pl.ds(0, G)], wbuf.at[p],
                                  isem[p]).wait()

        def issue_gathers(p):
            for k in range(4):
                pltpu.async_copy(xt_hbm.at[ibuf.at[p, k]], rows.at[p, k],
                                 gsem[p])

        def drain_gathers(p):
            for k in range(4):
                pltpu.make_async_copy(xt_hbm.at[pl.ds(0, G)], rows.at[p, k],
                                      gsem[p]).wait()

        def issue_out(j, p):
            base = wid * PPW + j * G
            pltpu.async_copy(obuf.at[p], out_hbm.at[pl.ds(base, G)], osem[p])

        def drain_out(p):
            pltpu.make_async_copy(obuf.at[p], out_hbm.at[pl.ds(0, G)],
                                  osem[p]).wait()

        def blend(p):
            @pl.loop(0, G)
            def _pix(i):
                isplat = lax.broadcast(i, (L,))
                wb = [plsc.load_gather(wbuf.at[p, k], [isplat])
                      for k in range(4)]
                for c0 in range(0, C, L):
                    acc = wb[0] * rows[p, 0, i, pl.ds(c0, L)]
                    acc = acc + wb[1] * rows[p, 1, i, pl.ds(c0, L)]
                    acc = acc + wb[2] * rows[p, 2, i, pl.ds(c0, L)]
                    acc = acc + wb[3] * rows[p, 3, i, pl.ds(c0, L)]
                    obuf[p, i, pl.ds(c0, L)] = acc

        issue_iw(0, 0)
        issue_iw(1, 1)
        drain_iw(0)
        issue_gathers(0)

        @pl.loop(0, CHUNKS, step=2)
        def _chunk(j2):
            for p in (0, 1):
                j = j2 + p
                q = 1 - p
                drain_gathers(p)

                @pl.when(j + 1 < CHUNKS)
                def _():
                    drain_iw(q)
                    issue_gathers(q)

                @pl.when(j >= 2)
                def _():
                    drain_out(p)

                blend(p)
                issue_out(j, p)

                @pl.when(j + 2 < CHUNKS)
                def _():
                    issue_iw(j + 2, p)

        drain_out(0)
        drain_out(1)

    return _sc_warp


def kernel(x, flow):
    assert x.shape == (N, C, H, W) and flow.shape == (N, 2, H, W)
    idx, wts, maskarr = _prep(flow)
    xt = jnp.concatenate(
        [jnp.transpose(x, (0, 2, 3, 1)),
         jnp.zeros((N, H, W, CP - C), jnp.float32)], axis=3)
    out_flat = _make_sc_warp()(xt.reshape(NPIX, CP), idx.reshape(4, NPIX),
                               wts.reshape(4, NPIX))
    warped = jnp.transpose(out_flat.reshape(N, H, W, CP)[..., :C], (0, 3, 1, 2))
    mask = maskarr.reshape(N, 1, H, W).astype(jnp.bool_)
    return warped, mask


# speedup vs baseline: 1.3322x; 1.0200x over previous
import dataclasses
import functools

import jax
import jax.numpy as jnp
from jax import lax
from jax.experimental import pallas as pl
from jax.experimental.pallas import tpu as pltpu
from jax.experimental.pallas import tpu_sc as plsc

N, C, H, W = 4, 96, 384, 384
CP = 128
NPIX = N * H * W
HB = 64
NC, NS, L = 2, 16, 16
NWORK = NC * NS
PPW = NPIX // NWORK
G = 96
CHUNKS = PPW // G


def _prep_body(flow_ref, idx_ref, w_ref, mask_ref):
    n = pl.program_id(0)
    hb = pl.program_id(1)
    fx = flow_ref[0, 0]
    fy = flow_ref[0, 1]
    wc = lax.broadcasted_iota(jnp.int32, (HB, W), 1).astype(jnp.float32)
    hc = (lax.broadcasted_iota(jnp.int32, (HB, W), 0) + hb * HB).astype(jnp.float32)
    gridx = wc + fx
    gridy = hc + fy
    gx = 2.0 * gridx / (W - 1) - 1.0
    gy = 2.0 * gridy / (H - 1) - 1.0
    ix = ((gx + 1.0) * W - 1.0) / 2.0
    iy = ((gy + 1.0) * H - 1.0) / 2.0
    ix0 = jnp.floor(ix)
    iy0 = jnp.floor(iy)
    ix1 = ix0 + 1.0
    iy1 = iy0 + 1.0
    wx1 = ix - ix0
    wx0 = 1.0 - wx1
    wy1 = iy - iy0
    wy0 = 1.0 - wy1
    mask = (jnp.abs(gx) <= 1.0) & (jnp.abs(gy) <= 1.0)
    mask_ref[...] = mask.astype(jnp.int32)
    corners = [(iy0, ix0, wy0 * wx0), (iy0, ix1, wy0 * wx1),
               (iy1, ix0, wy1 * wx0), (iy1, ix1, wy1 * wx1)]
    for k, (iyc, ixc, wgt) in enumerate(corners):
        valid = (ixc >= 0) & (ixc <= W - 1) & (iyc >= 0) & (iyc <= H - 1)
        ixs = jnp.clip(ixc, 0, W - 1).astype(jnp.int32)
        iys = jnp.clip(iyc, 0, H - 1).astype(jnp.int32)
        p = (n * H + iys) * W + ixs
        p = jnp.clip(p, 0, NPIX - 1)
        idx_ref[k] = p
        w_ref[k] = wgt * valid.astype(jnp.float32)


_prep = pl.pallas_call(
    _prep_body,
    grid=(N, H // HB),
    in_specs=[pl.BlockSpec((1, 2, HB, W), lambda n, hb: (n, 0, hb, 0))],
    out_specs=[
        pl.BlockSpec((4, HB, W), lambda n, hb: (0, n * (H // HB) + hb, 0)),
        pl.BlockSpec((4, HB, W), lambda n, hb: (0, n * (H // HB) + hb, 0)),
        pl.BlockSpec((HB, W), lambda n, hb: (n * (H // HB) + hb, 0)),
    ],
    out_shape=[
        jax.ShapeDtypeStruct((4, N * H, W), jnp.int32),
        jax.ShapeDtypeStruct((4, N * H, W), jnp.float32),
        jax.ShapeDtypeStruct((N * H, W), jnp.int32),
    ],
)


@functools.cache
def _make_sc_warp():
    cp = pltpu.CompilerParams()
    if "needs_layout_passes" in pltpu.CompilerParams.__dataclass_fields__:
        cp = dataclasses.replace(cp, needs_layout_passes=False)
    if "use_tc_tiling_on_sc" in pltpu.CompilerParams.__dataclass_fields__:
        cp = dataclasses.replace(cp, use_tc_tiling_on_sc=False)

    @functools.partial(
        pl.kernel,
        compiler_params=cp,
        out_type=jax.ShapeDtypeStruct((NPIX, CP), jnp.float32),
        mesh=plsc.VectorSubcoreMesh(core_axis_name="c", subcore_axis_name="s",
                                    num_cores=NC, num_subcores=NS),
        scratch_types=[
            pltpu.VMEM((2, 4, G), jnp.int32),
            pltpu.VMEM((2, 4, G), jnp.float32),
            pltpu.VMEM((2, 4, G, CP), jnp.float32),
            pltpu.VMEM((2, G, CP), jnp.float32),
            pltpu.SemaphoreType.DMA,
            pltpu.SemaphoreType.DMA,
            pltpu.SemaphoreType.DMA,
            pltpu.SemaphoreType.DMA,
            pltpu.SemaphoreType.DMA,
            pltpu.SemaphoreType.DMA,
        ],
    )
    def _sc_warp(xt_hbm, idx_hbm, w_hbm, out_hbm, ibuf, wbuf, rows, obuf,
                 g0, g1, i0, i1, o0, o1):
        cid = lax.axis_index("c")
        sid = lax.axis_index("s")
        wid = cid * NS + sid
        gsem = (g0, g1)
        isem = (i0, i1)
        osem = (o0, o1)

        def issue_iw(j, p):
            base = wid * PPW + j * G
            pltpu.async_copy(idx_hbm.at[:, pl.ds(base, G)], ibuf.at[p], isem[p])
            pltpu.async_copy(w_hbm.at[:, pl.ds(base, G)], wbuf.at[p], isem[p])

        def drain_iw(p):
            pltpu.make_async_copy(idx_hbm.at[:, pl.ds(0, G)], ibuf.at[p],
                                  isem[p]).wait()
            pltpu.make_async_copy(w_hbm.at[:, pl.ds(0, G)], wbuf.at[p],
                                  isem[p]).wait()

        def issue_gathers(p):
            for k in range(4):
                pltpu.async_copy(xt_hbm.at[ibuf.at[p, k]], rows.at[p, k],
                                 gsem[p])

        def drain_gathers(p):
            for k in range(4):
                pltpu.make_async_copy(xt_hbm.at[pl.ds(0, G)], rows.at[p, k],
                                      gsem[p]).wait()

        def issue_out(j, p):
            base = wid * PPW + j * G
            pltpu.async_copy(obuf.at[p], out_hbm.at[pl.ds(base, G)], osem[p])

        def drain_out(p):
            pltpu.make_async_copy(obuf.at[p], out_hbm.at[pl.ds(0, G)],
                                  osem[p]).wait()

        def blend(p):
            @pl.loop(0, G)
            def _pix(i):
                isplat = lax.broadcast(i, (L,))
                wb = [plsc.load_gather(wbuf.at[p, k], [isplat])
                      for k in range(4)]
                for c0 in range(0, C, L):
                    acc = wb[0] * rows[p, 0, i, pl.ds(c0, L)]
                    acc = acc + wb[1] * rows[p, 1, i, pl.ds(c0, L)]
                    acc = acc + wb[2] * rows[p, 2, i, pl.ds(c0, L)]
                    acc = acc + wb[3] * rows[p, 3, i, pl.ds(c0, L)]
                    obuf[p, i, pl.ds(c0, L)] = acc

        issue_iw(0, 0)
        issue_iw(1, 1)
        drain_iw(0)
        issue_gathers(0)

        @pl.loop(0, CHUNKS, step=2)
        def _chunk(j2):
            for p in (0, 1):
                j = j2 + p
                q = 1 - p

                @pl.when(j + 1 < CHUNKS)
                def _():
                    drain_iw(q)
                    issue_gathers(q)

                drain_gathers(p)

                @pl.when(j >= 2)
                def _():
                    drain_out(p)

                blend(p)
                issue_out(j, p)

                @pl.when(j + 2 < CHUNKS)
                def _():
                    issue_iw(j + 2, p)

        drain_out(0)
        drain_out(1)

    return _sc_warp


def kernel(x, flow):
    assert x.shape == (N, C, H, W) and flow.shape == (N, 2, H, W)
    idx, wts, maskarr = _prep(flow)
    xt = jnp.concatenate(
        [jnp.transpose(x, (0, 2, 3, 1)),
         jnp.zeros((N, H, W, CP - C), jnp.float32)], axis=3)
    out_flat = _make_sc_warp()(xt.reshape(NPIX, CP), idx.reshape(4, NPIX),
                               wts.reshape(4, NPIX))
    warped = jnp.transpose(out_flat.reshape(N, H, W, CP)[..., :C], (0, 3, 1, 2))
    mask = maskarr.reshape(N, 1, H, W).astype(jnp.bool_)
    return warped, mask
